# Initial kernel scaffold; baseline (speedup 1.0000x reference)
import functools

import jax
import jax.numpy as jnp
from jax import lax
from jax.experimental import pallas as pl
from jax.experimental.pallas import tpu as pltpu
from jax.experimental.pallas import tpu_sc as plsc

NUM_ROWS = 1_000_000
EMBED_DIM = 32
BATCH = 16384
HIST_LEN = 50
TOTAL = BATCH * HIST_LEN
NUM_WORKERS = 32
GATHER = 128
PER_WORKER = TOTAL // NUM_WORKERS
NUM_GATHERS = PER_WORKER // GATHER
IDX_ROWS = TOTAL // GATHER


_mesh = plsc.VectorSubcoreMesh(core_axis_name="c", subcore_axis_name="s")


@functools.partial(
    pl.kernel,
    mesh=_mesh,
    out_type=jax.ShapeDtypeStruct((TOTAL, EMBED_DIM), jnp.float32),
    scratch_types=[
        pltpu.VMEM((NUM_GATHERS, GATHER), jnp.int32),
        pltpu.VMEM((GATHER, EMBED_DIM), jnp.float32),
        pltpu.SemaphoreType.DMA,
    ],
)
def _embed_sc(idx_hbm, table_hbm, out_hbm, idx_v, rows_v, sem):
    wid = lax.axis_index("s") * 2 + lax.axis_index("c")
    idx_row_base = wid * NUM_GATHERS
    out_base = wid * PER_WORKER

    pltpu.sync_copy(idx_hbm.at[pl.ds(idx_row_base, NUM_GATHERS)], idx_v)

    def body(j, carry):
        pltpu.async_copy(table_hbm.at[idx_v.at[j]], rows_v, sem).wait()
        pltpu.sync_copy(rows_v, out_hbm.at[pl.ds(out_base + j * GATHER, GATHER)])
        return carry

    lax.fori_loop(0, NUM_GATHERS, body, 0)


def kernel(input, weight):
    idx = input.reshape(IDX_ROWS, GATHER)
    out = _embed_sc(idx, weight)
    return out.reshape(BATCH, HIST_LEN, EMBED_DIM)


# speedup vs baseline: 1.0238x; 1.0238x over previous
import functools

import jax
import jax.numpy as jnp
from jax import lax
from jax.experimental import pallas as pl
from jax.experimental.pallas import tpu as pltpu
from jax.experimental.pallas import tpu_sc as plsc

NUM_ROWS = 1_000_000
EMBED_DIM = 32
BATCH = 16384
HIST_LEN = 50
TOTAL = BATCH * HIST_LEN
NUM_WORKERS = 32
GATHER = 128
PER_WORKER = TOTAL // NUM_WORKERS
NUM_GATHERS = PER_WORKER // GATHER
IDX_ROWS = TOTAL // GATHER


_mesh = plsc.VectorSubcoreMesh(core_axis_name="c", subcore_axis_name="s")


@functools.partial(
    pl.kernel,
    mesh=_mesh,
    out_type=jax.ShapeDtypeStruct((TOTAL, EMBED_DIM), jnp.float32),
    scratch_types=[
        pltpu.VMEM((NUM_GATHERS, GATHER), jnp.int32),
        pltpu.VMEM((GATHER, EMBED_DIM), jnp.float32),
        pltpu.SemaphoreType.DMA,
    ],
    compiler_params=pltpu.CompilerParams(use_tc_tiling_on_sc=False),
)
def _embed_sc(idx_hbm, table_hbm, out_hbm, idx_v, rows_v, sem):
    wid = lax.axis_index("s") * 2 + lax.axis_index("c")
    idx_row_base = wid * NUM_GATHERS
    out_base = wid * PER_WORKER

    pltpu.sync_copy(idx_hbm.at[pl.ds(idx_row_base, NUM_GATHERS)], idx_v)

    def body(j, carry):
        pltpu.async_copy(table_hbm.at[idx_v.at[j]], rows_v, sem).wait()
        pltpu.sync_copy(rows_v, out_hbm.at[pl.ds(out_base + j * GATHER, GATHER)])
        return carry

    lax.fori_loop(0, NUM_GATHERS, body, 0)


def kernel(input, weight):
    idx = input.reshape(IDX_ROWS, GATHER)
    out = _embed_sc(idx, weight)
    return out.reshape(BATCH, HIST_LEN, EMBED_DIM)


# speedup vs baseline: 1.1125x; 1.0867x over previous
import functools

import jax
import jax.numpy as jnp
from jax import lax
from jax.experimental import pallas as pl
from jax.experimental.pallas import tpu as pltpu
from jax.experimental.pallas import tpu_sc as plsc

NUM_ROWS = 1_000_000
EMBED_DIM = 32
BATCH = 16384
HIST_LEN = 50
TOTAL = BATCH * HIST_LEN
NUM_WORKERS = 32
GATHER = 128
PER_WORKER = TOTAL // NUM_WORKERS
NUM_GATHERS = PER_WORKER // GATHER
IDX_ROWS = TOTAL // GATHER
NBUF = 8
NSTEP = NUM_GATHERS // NBUF


_mesh = plsc.VectorSubcoreMesh(core_axis_name="c", subcore_axis_name="s")

_scratch = (
    [pltpu.VMEM((NUM_GATHERS, GATHER), jnp.int32)]
    + [pltpu.VMEM((GATHER, EMBED_DIM), jnp.float32) for _ in range(NBUF)]
    + [pltpu.SemaphoreType.DMA for _ in range(2 * NBUF)]
)


@functools.partial(
    pl.kernel,
    mesh=_mesh,
    out_type=jax.ShapeDtypeStruct((TOTAL, EMBED_DIM), jnp.float32),
    scratch_types=_scratch,
    compiler_params=pltpu.CompilerParams(use_tc_tiling_on_sc=False),
)
def _embed_sc(idx_hbm, table_hbm, out_hbm, idx_v, *bufs_and_sems):
    rows = bufs_and_sems[:NBUF]
    sem_g = bufs_and_sems[NBUF : 2 * NBUF]
    sem_o = bufs_and_sems[2 * NBUF :]

    wid = lax.axis_index("s") * 2 + lax.axis_index("c")
    idx_row_base = wid * NUM_GATHERS
    out_base = wid * PER_WORKER

    pltpu.sync_copy(idx_hbm.at[pl.ds(idx_row_base, NUM_GATHERS)], idx_v)

    def gather(j, b):
        pltpu.async_copy(table_hbm.at[idx_v.at[j]], rows[b], sem_g[b])

    def store(j, b):
        pltpu.async_copy(
            rows[b], out_hbm.at[pl.ds(out_base + j * GATHER, GATHER)], sem_o[b]
        )

    def wait_gather(b):
        pltpu.make_async_copy(table_hbm.at[idx_v.at[0]], rows[b], sem_g[b]).wait()

    def wait_store(b):
        pltpu.make_async_copy(
            rows[b], out_hbm.at[pl.ds(out_base, GATHER)], sem_o[b]
        ).wait()

    for b in range(NBUF):
        gather(b, b)

    def body(g, carry):
        base_j = g * NBUF
        for b in range(NBUF):
            wait_gather(b)
            store(base_j + b, b)
        for b in range(NBUF):
            wait_store(b)
            gather(base_j + NBUF + b, b)
        return carry

    lax.fori_loop(0, NSTEP - 1, body, 0)

    base_j = (NSTEP - 1) * NBUF
    for b in range(NBUF):
        wait_gather(b)
        store(base_j + b, b)
    for b in range(NBUF):
        wait_store(b)


def kernel(input, weight):
    idx = input.reshape(IDX_ROWS, GATHER)
    out = _embed_sc(idx, weight)
    return out.reshape(BATCH, HIST_LEN, EMBED_DIM)


# speedup vs baseline: 1.5074x; 1.3549x over previous
import functools

import jax
import jax.numpy as jnp
from jax import lax
from jax.experimental import pallas as pl
from jax.experimental.pallas import tpu as pltpu
from jax.experimental.pallas import tpu_sc as plsc

NUM_ROWS = 1_000_000
EMBED_DIM = 32
BATCH = 16384
HIST_LEN = 50
NUM_WORKERS = 32
BLK = 128
NBLK = BATCH // BLK
BLK_PER_W = NBLK // NUM_WORKERS
BPW = BLK * BLK_PER_W
NBUF = 4


_mesh = plsc.VectorSubcoreMesh(core_axis_name="c", subcore_axis_name="s")

_scratch = (
    [pltpu.VMEM((HIST_LEN, BPW), jnp.int32)]
    + [pltpu.VMEM((BLK, EMBED_DIM), jnp.float32) for _ in range(NBUF)]
    + [pltpu.VMEM((4, 8, BLK), jnp.float32) for _ in range(NBUF)]
    + [pltpu.SemaphoreType.DMA for _ in range(2 * NBUF + 1)]
)


@functools.partial(
    pl.kernel,
    mesh=_mesh,
    out_type=jax.ShapeDtypeStruct((HIST_LEN, 4, NBLK, 8, BLK), jnp.float32),
    scratch_types=_scratch,
    compiler_params=pltpu.CompilerParams(
        use_tc_tiling_on_sc=False, needs_layout_passes=False
    ),
)
def _embed_sc(idx_hbm, table_hbm, out_hbm, idx_v, *bufs_and_sems):
    gbuf = bufs_and_sems[:NBUF]
    tbuf = bufs_and_sems[NBUF : 2 * NBUF]
    sem_g = bufs_and_sems[2 * NBUF : 3 * NBUF]
    sem_o = bufs_and_sems[3 * NBUF : 4 * NBUF]
    sem_i = bufs_and_sems[4 * NBUF]

    wid = lax.axis_index("s") * 2 + lax.axis_index("c")
    b_base = wid * BPW
    c_base = wid * BLK_PER_W

    pltpu.async_copy(idx_hbm.at[:, pl.ds(b_base, BPW)], idx_v, sem_i).wait()

    def gather(h, j):
        idx = idx_v.at[h, pl.ds(j * BLK, BLK)]
        pltpu.async_copy(table_hbm.at[idx], gbuf[j], sem_g[j])

    def store(h, j):
        pltpu.async_copy(tbuf[j], out_hbm.at[h, :, c_base + j, :, :], sem_o[j])

    def wait_gather(j):
        idx = idx_v.at[0, pl.ds(0, BLK)]
        pltpu.make_async_copy(table_hbm.at[idx], gbuf[j], sem_g[j]).wait()

    def wait_store(j):
        pltpu.make_async_copy(
            tbuf[j], out_hbm.at[0, :, 0, :, :], sem_o[j]
        ).wait()

    lanes = jax.lax.iota(jnp.int32, 16)
    rowsets = [lanes + c0 for c0 in range(0, BLK, 16)]

    def transpose(j):
        for f in range(EMBED_DIM):
            col = jnp.full((16,), f, jnp.int32)
            for ci, c0 in enumerate(range(0, BLK, 16)):
                v = plsc.load_gather(gbuf[j], [rowsets[ci], col])
                tbuf[j][f // 8, f % 8, pl.ds(c0, 16)] = v

    for j in range(NBUF):
        gather(0, j)

    def body(h, carry):
        for j in range(NBUF):
            wait_gather(j)

            @pl.when(h > 0)
            def _():
                wait_store(j)

            transpose(j)

            @pl.when(h < HIST_LEN - 1)
            def _():
                gather(h + 1, j)

            store(h, j)
        return carry

    lax.fori_loop(0, HIST_LEN, body, 0)

    for j in range(NBUF):
        wait_store(j)


def kernel(input, weight):
    out5 = _embed_sc(input.T, weight)
    return out5.transpose(2, 4, 0, 1, 3).reshape(BATCH, HIST_LEN, EMBED_DIM)


# speedup vs baseline: 2.5680x; 1.7036x over previous
import functools

import jax
import jax.numpy as jnp
from jax import lax
from jax.experimental import pallas as pl
from jax.experimental.pallas import tpu as pltpu
from jax.experimental.pallas import tpu_sc as plsc

NUM_ROWS = 1_000_000
EMBED_DIM = 32
BATCH = 16384
HIST_LEN = 50
NUM_WORKERS = 32
BLK = 128
NBLK = BATCH // BLK
BLK_PER_W = NBLK // NUM_WORKERS
BPW = BLK * BLK_PER_W
NBUF = 4


_mesh = plsc.VectorSubcoreMesh(core_axis_name="c", subcore_axis_name="s")

_scratch = (
    [pltpu.VMEM((HIST_LEN, BPW), jnp.int32)]
    + [pltpu.VMEM((BLK, EMBED_DIM), jnp.float32) for _ in range(NBUF)]
    + [pltpu.VMEM((4, 8, BLK + 1), jnp.float32) for _ in range(NBUF)]
    + [pltpu.SemaphoreType.DMA for _ in range(2 * NBUF + 1)]
)


@functools.partial(
    pl.kernel,
    mesh=_mesh,
    out_type=jax.ShapeDtypeStruct((HIST_LEN, 4, NBLK, 8, BLK), jnp.float32),
    scratch_types=_scratch,
    compiler_params=pltpu.CompilerParams(
        use_tc_tiling_on_sc=False, needs_layout_passes=False
    ),
)
def _embed_sc(idx_hbm, table_hbm, out_hbm, idx_v, *bufs_and_sems):
    gbuf = bufs_and_sems[:NBUF]
    tbuf = bufs_and_sems[NBUF : 2 * NBUF]
    sem_g = bufs_and_sems[2 * NBUF : 3 * NBUF]
    sem_o = bufs_and_sems[3 * NBUF : 4 * NBUF]
    sem_i = bufs_and_sems[4 * NBUF]

    wid = lax.axis_index("s") * 2 + lax.axis_index("c")
    b_base = wid * BPW
    c_base = wid * BLK_PER_W

    pltpu.async_copy(idx_hbm.at[:, pl.ds(b_base, BPW)], idx_v, sem_i).wait()

    def gather(h, j):
        idx = idx_v.at[h, pl.ds(j * BLK, BLK)]
        pltpu.async_copy(table_hbm.at[idx], gbuf[j], sem_g[j])

    def store(h, j):
        pltpu.async_copy(
            tbuf[j].at[:, :, pl.ds(0, BLK)],
            out_hbm.at[h, :, c_base + j, :, :],
            sem_o[j],
        )

    def wait_gather(j):
        idx = idx_v.at[0, pl.ds(0, BLK)]
        pltpu.make_async_copy(table_hbm.at[idx], gbuf[j], sem_g[j]).wait()

    def wait_store(j):
        pltpu.make_async_copy(
            tbuf[j].at[:, :, pl.ds(0, BLK)], out_hbm.at[0, :, 0, :, :], sem_o[j]
        ).wait()

    lanes = jax.lax.iota(jnp.int32, 16)
    fcoords = [
        ((lanes + f0) // 8, (lanes + f0) % 8) for f0 in range(0, EMBED_DIM, 16)
    ]

    def transpose(j):
        def trow(b, carry):
            col = jnp.full((16,), b, jnp.int32)
            for fi, f0 in enumerate(range(0, EMBED_DIM, 16)):
                v = gbuf[j][b, pl.ds(f0, 16)]
                plsc.store_scatter(tbuf[j], [fcoords[fi][0], fcoords[fi][1], col], v)
            return carry

        lax.fori_loop(0, BLK, trow, 0)

    for j in range(NBUF):
        gather(0, j)

    def body(h, carry):
        for j in range(NBUF):
            wait_gather(j)

            @pl.when(h > 0)
            def _():
                wait_store(j)

            transpose(j)

            @pl.when(h < HIST_LEN - 1)
            def _():
                gather(h + 1, j)

            store(h, j)
        return carry

    lax.fori_loop(0, HIST_LEN, body, 0)

    for j in range(NBUF):
        wait_store(j)


def kernel(input, weight):
    out5 = _embed_sc(input.T, weight)
    return out5.transpose(2, 4, 0, 1, 3).reshape(BATCH, HIST_LEN, EMBED_DIM)


# speedup vs baseline: 2.6410x; 1.0284x over previous
import functools

import jax
import jax.numpy as jnp
from jax import lax
from jax.experimental import pallas as pl
from jax.experimental.pallas import tpu as pltpu
from jax.experimental.pallas import tpu_sc as plsc

NUM_ROWS = 1_000_000
EMBED_DIM = 32
BATCH = 16384
HIST_LEN = 50
NUM_WORKERS = 32
BLK = 128
NBLK = BATCH // BLK
BLK_PER_W = NBLK // NUM_WORKERS
BPW = BLK * BLK_PER_W
NBUF = 8


_mesh = plsc.VectorSubcoreMesh(core_axis_name="c", subcore_axis_name="s")

_scratch = (
    [pltpu.VMEM((HIST_LEN, BPW), jnp.int32)]
    + [pltpu.VMEM((BLK, EMBED_DIM), jnp.float32) for _ in range(NBUF)]
    + [pltpu.VMEM((4, 8, BLK + 1), jnp.float32) for _ in range(NBUF)]
    + [pltpu.SemaphoreType.DMA for _ in range(2 * NBUF + 1)]
)


@functools.partial(
    pl.kernel,
    mesh=_mesh,
    out_type=jax.ShapeDtypeStruct((HIST_LEN, 4, NBLK, 8, BLK), jnp.float32),
    scratch_types=_scratch,
    compiler_params=pltpu.CompilerParams(
        use_tc_tiling_on_sc=False, needs_layout_passes=False
    ),
)
def _embed_sc(idx_hbm, table_hbm, out_hbm, idx_v, *bufs_and_sems):
    gbuf = bufs_and_sems[:NBUF]
    tbuf = bufs_and_sems[NBUF : 2 * NBUF]
    sem_g = bufs_and_sems[2 * NBUF : 3 * NBUF]
    sem_o = bufs_and_sems[3 * NBUF : 4 * NBUF]
    sem_i = bufs_and_sems[4 * NBUF]

    wid = lax.axis_index("s") * 2 + lax.axis_index("c")
    b_base = wid * BPW
    c_base = wid * BLK_PER_W

    pltpu.async_copy(idx_hbm.at[:, pl.ds(b_base, BPW)], idx_v, sem_i).wait()

    def gather(h, j, s):
        idx = idx_v.at[h, pl.ds(j * BLK, BLK)]
        pltpu.async_copy(table_hbm.at[idx], gbuf[s], sem_g[s])

    def store(h, j, s):
        pltpu.async_copy(
            tbuf[s].at[:, :, pl.ds(0, BLK)],
            out_hbm.at[h, :, c_base + j, :, :],
            sem_o[s],
        )

    def wait_gather(s):
        idx = idx_v.at[0, pl.ds(0, BLK)]
        pltpu.make_async_copy(table_hbm.at[idx], gbuf[s], sem_g[s]).wait()

    def wait_store(s):
        pltpu.make_async_copy(
            tbuf[s].at[:, :, pl.ds(0, BLK)], out_hbm.at[0, :, 0, :, :], sem_o[s]
        ).wait()

    lanes = jax.lax.iota(jnp.int32, 16)
    fcoords = [
        ((lanes + f0) // 8, (lanes + f0) % 8) for f0 in range(0, EMBED_DIM, 16)
    ]

    UNROLL = 4

    def transpose(s):
        def trow(t, carry):
            for u in range(UNROLL):
                b = t * UNROLL + u
                col = jnp.full((16,), b, jnp.int32)
                for fi, f0 in enumerate(range(0, EMBED_DIM, 16)):
                    v = gbuf[s][b, pl.ds(f0, 16)]
                    plsc.store_scatter(
                        tbuf[s], [fcoords[fi][0], fcoords[fi][1], col], v
                    )
            return carry

        lax.fori_loop(0, BLK // UNROLL, trow, 0)

    for s in range(NBUF):
        gather(s // BLK_PER_W, s % BLK_PER_W, s)

    def body(g, carry):
        h0 = 2 * g
        for s in range(NBUF):
            h = h0 + s // BLK_PER_W
            j = s % BLK_PER_W
            wait_gather(s)

            @pl.when(g > 0)
            def _():
                wait_store(s)

            transpose(s)

            @pl.when(h < HIST_LEN - 2)
            def _():
                gather(h + 2, j, s)

            store(h, j, s)
        return carry

    lax.fori_loop(0, HIST_LEN // 2, body, 0)

    for s in range(NBUF):
        wait_store(s)


def kernel(input, weight):
    out5 = _embed_sc(input.T, weight)
    return out5.transpose(2, 4, 0, 1, 3).reshape(BATCH, HIST_LEN, EMBED_DIM)
